# baseline (speedup 1.0000x reference)
import functools

import jax
import jax.numpy as jnp
from jax import lax
from jax.experimental import pallas as pl
from jax.experimental.pallas import tpu as pltpu
from jax.experimental.pallas import tpu_sc as plsc

_BATCH = 16384
_HIST = 50
_EMBED = 32
_NW = 32
_BATCH_W = _BATCH // _NW
_Q = 128
_NQ = _BATCH_W // _Q
_STEPS = _HIST * _NQ
_VOCAB = 1000000
_VW = _VOCAB // 4


@functools.partial(
    pl.kernel,
    mesh=plsc.VectorSubcoreMesh(core_axis_name="c", subcore_axis_name="s"),
    out_type=jax.ShapeDtypeStruct((_HIST, _EMBED, _BATCH), jnp.float32),
    compiler_params=pltpu.CompilerParams(
        needs_layout_passes=False, use_tc_tiling_on_sc=False),
    scratch_types=[
        pltpu.VMEM((_HIST, _BATCH_W), jnp.int32),
        pltpu.VMEM((_BATCH_W,), jnp.int32),
        pltpu.VMEM((_Q,), jnp.int32),
        pltpu.VMEM((_Q,), jnp.int32),
        pltpu.VMEM((_Q,), jnp.int32),
        pltpu.VMEM((_Q,), jnp.int32),
        pltpu.VMEM((_Q, _EMBED), jnp.float32),
        pltpu.VMEM((_Q, _EMBED), jnp.float32),
        pltpu.VMEM((_Q, _EMBED), jnp.float32),
        pltpu.VMEM((_Q, _EMBED), jnp.float32),
        pltpu.VMEM((_EMBED, _BATCH_W), jnp.float32),
        pltpu.SemaphoreType.DMA,
        pltpu.SemaphoreType.DMA,
        pltpu.SemaphoreType.DMA,
        pltpu.SemaphoreType.DMA,
        pltpu.SemaphoreType.DMA,
    ],
)
def _emb_lookup(idst_hbm, len_hbm, tablew_hbm, outt_hbm,
                idst_v, len_v, idx0, idx1, idx2, idx3,
                data0, data1, data2, data3,
                stage, gsem0, gsem1, gsem2, gsem3, osem):
    wid = lax.axis_index("s") * 2 + lax.axis_index("c")
    b0 = wid * _BATCH_W
    pltpu.sync_copy(idst_hbm.at[:, pl.ds(b0, _BATCH_W)], idst_v)
    pltpu.sync_copy(len_hbm.at[pl.ds(b0, _BATCH_W)], len_v)
    iota = lax.iota(jnp.int32, 16)
    idxs = (idx0, idx1, idx2, idx3)
    datas = (data0, data1, data2, data3)
    gsems = (gsem0, gsem1, gsem2, gsem3)

    def build_idx(t, p):
        j = t >> 2
        c0 = (t & 3) * _Q
        for ib in range(_Q // 16):
            iv = idst_v[j, pl.ds(c0 + ib * 16, 16)]
            idxs[p][pl.ds(ib * 16, 16)] = iv

    def start_gather(p):
        pltpu.async_copy(tablew_hbm.at[idxs[p]], datas[p], gsems[p])

    def wait_gather(p):
        pltpu.make_async_copy(tablew_hbm.at[idxs[p]], datas[p],
                              gsems[p]).wait()

    for p in range(3):
        build_idx(p, p)
        start_gather(p)

    def jstep(j, carry):
        @pl.when(j >= 1)
        def _():
            pltpu.make_async_copy(
                stage, outt_hbm.at[j - 1, :, pl.ds(b0, _BATCH_W)],
                osem).wait()

        for q in range(4):
            t = j * 4 + q
            p = q

            wait_gather(p)

            @pl.when(t + 3 < _STEPS)
            def _(t=t, q=q):
                build_idx(t + 3, (q + 3) & 3)
                start_gather((q + 3) & 3)
            dv = datas[p]
            c0 = q * _Q
            for ib in range(_Q // 16):
                rows = iota + ib * 16
                lenv = len_v[pl.ds(c0 + ib * 16, 16)]
                keep = lenv > j
                for d in range(_EMBED):
                    g = plsc.load_gather(
                        dv, [rows, jnp.full((16,), d, jnp.int32)])
                    stage[d, pl.ds(c0 + ib * 16, 16)] = jnp.where(keep, g, 0.0)

        pltpu.async_copy(stage, outt_hbm.at[j, :, pl.ds(b0, _BATCH_W)],
                         osem)
        return carry

    lax.fori_loop(0, _HIST, jstep, 0)
    pltpu.make_async_copy(stage, outt_hbm.at[_HIST - 1, :,
                                             pl.ds(b0, _BATCH_W)],
                          osem).wait()


def kernel(ids, length, embedding_table):
    out_t = _emb_lookup(ids.T, length, embedding_table)
    return out_t.transpose(2, 0, 1)


# speedup vs baseline: 1.1493x; 1.1493x over previous
import functools

import jax
import jax.numpy as jnp
from jax import lax
from jax.experimental import pallas as pl
from jax.experimental.pallas import tpu as pltpu
from jax.experimental.pallas import tpu_sc as plsc

_BATCH = 16384
_HIST = 50
_EMBED = 32
_NW = 32
_BATCH_W = _BATCH // _NW
_Q = 128
_NQ = _BATCH_W // _Q
_STEPS = _HIST * _NQ
_VOCAB = 1000000
_VW = _VOCAB // 4


@functools.partial(
    pl.kernel,
    mesh=plsc.VectorSubcoreMesh(core_axis_name="c", subcore_axis_name="s"),
    out_type=jax.ShapeDtypeStruct((_HIST, _EMBED, _BATCH), jnp.float32),
    compiler_params=pltpu.CompilerParams(needs_layout_passes=False),
    scratch_types=[
        pltpu.VMEM((_HIST, _BATCH_W), jnp.int32),
        pltpu.VMEM((_BATCH_W,), jnp.int32),
        pltpu.VMEM((_Q,), jnp.int32),
        pltpu.VMEM((_Q,), jnp.int32),
        pltpu.VMEM((_Q,), jnp.int32),
        pltpu.VMEM((_Q,), jnp.int32),
        pltpu.VMEM((_Q, 128), jnp.float32),
        pltpu.VMEM((_Q, 128), jnp.float32),
        pltpu.VMEM((_EMBED, _Q), jnp.float32),
        pltpu.SemaphoreType.DMA,
        pltpu.SemaphoreType.DMA,
    ],
)
def _emb_lookup(idst_hbm, len_hbm, tablew_hbm, outt_hbm,
                idst_v, len_v, idx0, idx1, colb0, colb1, data0, data1,
                stage_v, gsem0, gsem1):
    wid = lax.axis_index("s") * 2 + lax.axis_index("c")
    b0 = wid * _BATCH_W
    pltpu.sync_copy(idst_hbm.at[:, pl.ds(b0, _BATCH_W)], idst_v)
    pltpu.sync_copy(len_hbm.at[pl.ds(b0, _BATCH_W)], len_v)
    iota = lax.iota(jnp.int32, 16)
    idxs = (idx0, idx1)
    colbs = (colb0, colb1)
    datas = (data0, data1)
    gsems = (gsem0, gsem1)

    def build_idx(t, idx_v, colb_v):
        j = t >> 2
        c0 = (t & 3) * _Q
        for ib in range(_Q // 16):
            iv = idst_v[j, pl.ds(c0 + ib * 16, 16)]
            idx_v[pl.ds(ib * 16, 16)] = iv >> 2
            colb_v[pl.ds(ib * 16, 16)] = (iv & 3) << 5

    def start_gather(p):
        pltpu.async_copy(tablew_hbm.at[idxs[p]], datas[p], gsems[p])

    def wait_gather(p):
        pltpu.make_async_copy(tablew_hbm.at[idxs[p]], datas[p],
                              gsems[p]).wait()

    build_idx(0, idx0, colb0)
    start_gather(0)

    def step(t, carry):
        p = t & 1

        @pl.when(t + 1 < _STEPS)
        def _():
            @pl.when(p == 0)
            def _():
                build_idx(t + 1, idx1, colb1)
                start_gather(1)

            @pl.when(p == 1)
            def _():
                build_idx(t + 1, idx0, colb0)
                start_gather(0)

        j = t >> 2
        c0 = (t & 3) * _Q

        def consume(p_static):
            wait_gather(p_static)
            dv = datas[p_static]
            cbv = colbs[p_static]
            for ib in range(_Q // 16):
                rows = iota + ib * 16
                lenv = len_v[pl.ds(c0 + ib * 16, 16)]
                keep = lenv > j
                colb = cbv[pl.ds(ib * 16, 16)]
                for d in range(_EMBED):
                    g = plsc.load_gather(dv, [rows, colb + d])
                    stage_v[d, pl.ds(ib * 16, 16)] = jnp.where(keep, g, 0.0)

        @pl.when(p == 0)
        def _():
            consume(0)

        @pl.when(p == 1)
        def _():
            consume(1)

        pltpu.sync_copy(stage_v, outt_hbm.at[j, :, pl.ds(b0 + c0, _Q)])
        return carry

    lax.fori_loop(0, _STEPS, step, 0)


def kernel(ids, length, embedding_table):
    out_t = _emb_lookup(ids.T, length, embedding_table.reshape(_VW, 128))
    return out_t.transpose(2, 0, 1)
